# Initial kernel scaffold; baseline (speedup 1.0000x reference)
import functools
import jax
import jax.numpy as jnp
from jax.experimental import pallas as pl
from jax.experimental.pallas import tpu as pltpu

B, D, N, K, FE = 4, 128, 1024, 16, 4
C_OUT, CE_OUT = 128, 16
BN = 128
NB = N // BN
Q = K * BN
M_ALL = B * N * K
EPS = 1e-5


def _stage1_body(f_ref, idxt_ref, eft_ref, W0_ref, We0_ref,
                 ymax_ref, zmax_ref, efmax_ref, ystats_ref, zstats_ref,
                 g_scr):
    b = pl.program_id(0)
    nb = pl.program_id(1)

    @pl.when(nb == 0)
    def _():
        f = f_ref[0]
        Wa = W0_ref[:, :D]
        Wb = W0_ref[:, D:2 * D]
        g1 = jnp.dot(Wa - Wb, f, preferred_element_type=jnp.float32)
        g2 = jnp.dot(Wb, f, preferred_element_type=jnp.float32)
        g_scr[0] = g1
        g_scr[1] = g2

    g2 = g_scr[1]
    g2_hi = g2.astype(jnp.bfloat16)
    g2_lo = (g2 - g2_hi.astype(jnp.float32)).astype(jnp.bfloat16)

    idxq = idxt_ref[0].reshape(1, Q)
    riota = jax.lax.broadcasted_iota(jnp.int32, (N, Q), 0)
    O = (riota == idxq).astype(jnp.bfloat16)

    y = jnp.dot(g2_hi, O, preferred_element_type=jnp.float32)
    y = y + jnp.dot(g2_lo, O, preferred_element_type=jnp.float32)

    ef = eft_ref[0].reshape(FE, Q)
    We = W0_ref[:, 2 * D:2 * D + FE]
    We0 = We0_ref[...]
    z = jnp.zeros((CE_OUT, Q), jnp.float32)
    for f in range(FE):
        efr = ef[f:f + 1, :]
        y = y + We[:, f:f + 1] * efr
        z = z + We0[:, f:f + 1] * efr

    g1blk = g_scr[0][:, nb * BN:(nb + 1) * BN]
    for j in range(K):
        y = y.at[:, j * BN:(j + 1) * BN].add(g1blk)

    ysum = jnp.sum(y, axis=1)
    ysq = jnp.sum(y * y, axis=1)
    zsum = jnp.sum(z, axis=1)
    zsq = jnp.sum(z * z, axis=1)

    ymax = y[:, :BN]
    zmaxv = z[:, :BN]
    efmax = ef[:, :BN]
    for j in range(1, K):
        sl = slice(j * BN, (j + 1) * BN)
        ymax = jnp.maximum(ymax, y[:, sl])
        zmaxv = jnp.maximum(zmaxv, z[:, sl])
        efmax = jnp.maximum(efmax, ef[:, sl])
    ymax_ref[0] = ymax
    zmax_ref[0] = zmaxv
    efmax_ref[0] = efmax

    first = jnp.logical_and(b == 0, nb == 0)

    @pl.when(first)
    def _():
        ystats_ref[0, :] = ysum
        ystats_ref[1, :] = ysq
        zstats_ref[0, :] = zsum
        zstats_ref[1, :] = zsq

    @pl.when(jnp.logical_not(first))
    def _():
        ystats_ref[0, :] += ysum
        ystats_ref[1, :] += ysq
        zstats_ref[0, :] += zsum
        zstats_ref[1, :] += zsq


def _stage2_body(f_ref, ymax_ref, zmax_ref, efmax_ref, ystats_ref,
                 zstats_ref, Wsc_ref, out_ref, outef_ref):
    my = ystats_ref[0, :] * (1.0 / M_ALL)
    vy = ystats_ref[1, :] * (1.0 / M_ALL) - my * my
    ry = jax.lax.rsqrt(vy + EPS)
    mz = zstats_ref[0, :] * (1.0 / M_ALL)
    vz = zstats_ref[1, :] * (1.0 / M_ALL) - mz * mz
    rz = jax.lax.rsqrt(vz + EPS)

    fts = jnp.maximum((ymax_ref[...] - my[None, :, None]) * ry[None, :, None], 0.0)
    out_ref[...] = jnp.maximum(f_ref[...] + fts, 0.0)

    Wsc = Wsc_ref[...]
    ssum = jnp.zeros((CE_OUT,), jnp.float32)
    ssq = jnp.zeros((CE_OUT,), jnp.float32)
    s_all = []
    for b in range(B):
        s = jnp.zeros((CE_OUT, N), jnp.float32)
        for f in range(FE):
            s = s + Wsc[:, f:f + 1] * efmax_ref[b, f:f + 1, :]
        s_all.append(s)
        ssum = ssum + jnp.sum(s, axis=1)
        ssq = ssq + jnp.sum(s * s, axis=1)
    ms = ssum * (1.0 / (B * N))
    vs = ssq * (1.0 / (B * N)) - ms * ms
    rs = jax.lax.rsqrt(vs + EPS)
    for b in range(B):
        fts_ef = jnp.maximum((zmax_ref[b] - mz[:, None]) * rz[:, None], 0.0)
        sc = (s_all[b] - ms[:, None]) * rs[:, None]
        outef_ref[b] = jnp.maximum(sc + fts_ef, 0.0)


def _dense_stages(features, idx_t, ef_t, W0, We0, Wsc_ef, interpret=False):
    ymax, zmax, efmax, ystats, zstats = pl.pallas_call(
        _stage1_body,
        grid=(B, NB),
        in_specs=[
            pl.BlockSpec((1, D, N), lambda b, nb: (b, 0, 0)),
            pl.BlockSpec((1, K, BN), lambda b, nb: (b, 0, nb)),
            pl.BlockSpec((1, FE, K, BN), lambda b, nb: (b, 0, 0, nb)),
            pl.BlockSpec((C_OUT, 2 * D + FE), lambda b, nb: (0, 0)),
            pl.BlockSpec((CE_OUT, FE), lambda b, nb: (0, 0)),
        ],
        out_specs=[
            pl.BlockSpec((1, C_OUT, BN), lambda b, nb: (b, 0, nb)),
            pl.BlockSpec((1, CE_OUT, BN), lambda b, nb: (b, 0, nb)),
            pl.BlockSpec((1, FE, BN), lambda b, nb: (b, 0, nb)),
            pl.BlockSpec((2, C_OUT), lambda b, nb: (0, 0)),
            pl.BlockSpec((2, CE_OUT), lambda b, nb: (0, 0)),
        ],
        out_shape=[
            jax.ShapeDtypeStruct((B, C_OUT, N), jnp.float32),
            jax.ShapeDtypeStruct((B, CE_OUT, N), jnp.float32),
            jax.ShapeDtypeStruct((B, FE, N), jnp.float32),
            jax.ShapeDtypeStruct((2, C_OUT), jnp.float32),
            jax.ShapeDtypeStruct((2, CE_OUT), jnp.float32),
        ],
        scratch_shapes=[pltpu.VMEM((2, D, N), jnp.float32)],
        interpret=interpret,
    )(features, idx_t, ef_t, W0, We0)

    out, out_ef = pl.pallas_call(
        _stage2_body,
        out_shape=[
            jax.ShapeDtypeStruct((B, C_OUT, N), jnp.float32),
            jax.ShapeDtypeStruct((B, CE_OUT, N), jnp.float32),
        ],
        interpret=interpret,
    )(features, ymax, zmax, efmax, ystats, zstats, Wsc_ef)
    return out, out_ef


def _ef_xla(edge_list, edge_features, idx):
    src = edge_list[:, 0, :]
    dst = edge_list[:, 1, :]
    key_e = src * N + dst
    eft = jnp.transpose(edge_features, (0, 2, 1))
    dense = jnp.zeros((B, N * N, FE), jnp.float32)
    dense = jax.vmap(lambda d, k, v: d.at[k].add(v))(dense, key_e, eft)
    key_q = jnp.arange(N)[None, :, None] * N + idx
    ef = jax.vmap(lambda d, k: d[k.reshape(-1)])(dense, key_q)
    ef = ef.reshape(B, N, K, FE)
    return jnp.transpose(ef, (0, 3, 2, 1))


@jax.jit
def kernel(points, features, edge_list, edge_features, idx, W0, We0, Wsc_ef):
    del points
    ef_t = _ef_xla(edge_list, edge_features, idx)
    idx_t = jnp.transpose(idx, (0, 2, 1))
    return _dense_stages(features, idx_t, ef_t, W0, We0, Wsc_ef)


# speedup vs baseline: 3.4668x; 3.4668x over previous
import functools
import jax
import jax.numpy as jnp
from jax.experimental import pallas as pl
from jax.experimental.pallas import tpu as pltpu

B, D, N, K, FE = 4, 128, 1024, 16, 4
C_OUT, CE_OUT = 128, 16
BN = 128
NB = N // BN
Q = K * BN
M_ALL = B * N * K
EPS = 1e-5


def _stage1_body(f_ref, idxt_ref, eft_ref, W0_ref, We0_ref,
                 ymax_ref, zmax_ref, efmax_ref, ystats_ref, zstats_ref,
                 g_scr):
    b = pl.program_id(0)
    nb = pl.program_id(1)

    @pl.when(nb == 0)
    def _():
        f = f_ref[0]
        Wa = W0_ref[:, :D]
        Wb = W0_ref[:, D:2 * D]
        g1 = jnp.dot(Wa - Wb, f, preferred_element_type=jnp.float32)
        g2 = jnp.dot(Wb, f, preferred_element_type=jnp.float32)
        g_scr[0] = g1
        g_scr[1] = g2

    g2 = g_scr[1]
    g2_hi = g2.astype(jnp.bfloat16)
    g2_lo = (g2 - g2_hi.astype(jnp.float32)).astype(jnp.bfloat16)

    idxq = idxt_ref[0].reshape(1, Q)
    riota = jax.lax.broadcasted_iota(jnp.int32, (N, Q), 0)
    O = (riota == idxq).astype(jnp.bfloat16)

    y = jnp.dot(g2_hi, O, preferred_element_type=jnp.float32)
    y = y + jnp.dot(g2_lo, O, preferred_element_type=jnp.float32)

    ef = eft_ref[0].reshape(FE, Q)
    We = W0_ref[:, 2 * D:2 * D + FE]
    We0 = We0_ref[...]
    z = jnp.zeros((CE_OUT, Q), jnp.float32)
    for f in range(FE):
        efr = ef[f:f + 1, :]
        y = y + We[:, f:f + 1] * efr
        z = z + We0[:, f:f + 1] * efr

    g1blk = g_scr[0, :, pl.ds(nb * BN, BN)]
    y = y + jnp.concatenate([g1blk] * K, axis=1)

    ysum = jnp.sum(y, axis=1)
    ysq = jnp.sum(y * y, axis=1)
    zsum = jnp.sum(z, axis=1)
    zsq = jnp.sum(z * z, axis=1)

    ymax = y[:, :BN]
    zmaxv = z[:, :BN]
    efmax = ef[:, :BN]
    for j in range(1, K):
        sl = slice(j * BN, (j + 1) * BN)
        ymax = jnp.maximum(ymax, y[:, sl])
        zmaxv = jnp.maximum(zmaxv, z[:, sl])
        efmax = jnp.maximum(efmax, ef[:, sl])
    ymax_ref[0] = ymax
    zmax_ref[0] = zmaxv
    efmax_ref[0] = efmax

    first = jnp.logical_and(b == 0, nb == 0)

    @pl.when(first)
    def _():
        ystats_ref[0, :] = ysum
        ystats_ref[1, :] = ysq
        zstats_ref[0, :] = zsum
        zstats_ref[1, :] = zsq

    @pl.when(jnp.logical_not(first))
    def _():
        ystats_ref[0, :] += ysum
        ystats_ref[1, :] += ysq
        zstats_ref[0, :] += zsum
        zstats_ref[1, :] += zsq


def _stage2_body(f_ref, ymax_ref, zmax_ref, efmax_ref, ystats_ref,
                 zstats_ref, Wsc_ref, out_ref, outef_ref):
    my = ystats_ref[0, :] * (1.0 / M_ALL)
    vy = ystats_ref[1, :] * (1.0 / M_ALL) - my * my
    ry = jax.lax.rsqrt(vy + EPS)
    mz = zstats_ref[0, :] * (1.0 / M_ALL)
    vz = zstats_ref[1, :] * (1.0 / M_ALL) - mz * mz
    rz = jax.lax.rsqrt(vz + EPS)

    fts = jnp.maximum((ymax_ref[...] - my[None, :, None]) * ry[None, :, None], 0.0)
    out_ref[...] = jnp.maximum(f_ref[...] + fts, 0.0)

    Wsc = Wsc_ref[...]
    ssum = jnp.zeros((CE_OUT,), jnp.float32)
    ssq = jnp.zeros((CE_OUT,), jnp.float32)
    s_all = []
    for b in range(B):
        s = jnp.zeros((CE_OUT, N), jnp.float32)
        for f in range(FE):
            s = s + Wsc[:, f:f + 1] * efmax_ref[b, f:f + 1, :]
        s_all.append(s)
        ssum = ssum + jnp.sum(s, axis=1)
        ssq = ssq + jnp.sum(s * s, axis=1)
    ms = ssum * (1.0 / (B * N))
    vs = ssq * (1.0 / (B * N)) - ms * ms
    rs = jax.lax.rsqrt(vs + EPS)
    for b in range(B):
        fts_ef = jnp.maximum((zmax_ref[b] - mz[:, None]) * rz[:, None], 0.0)
        sc = (s_all[b] - ms[:, None]) * rs[:, None]
        outef_ref[b] = jnp.maximum(sc + fts_ef, 0.0)


def _dense_stages(features, idx_t, ef_t, W0, We0, Wsc_ef, interpret=False):
    ymax, zmax, efmax, ystats, zstats = pl.pallas_call(
        _stage1_body,
        grid=(B, NB),
        in_specs=[
            pl.BlockSpec((1, D, N), lambda b, nb: (b, 0, 0)),
            pl.BlockSpec((1, K, BN), lambda b, nb: (b, 0, nb)),
            pl.BlockSpec((1, FE, K, BN), lambda b, nb: (b, 0, 0, nb)),
            pl.BlockSpec((C_OUT, 2 * D + FE), lambda b, nb: (0, 0)),
            pl.BlockSpec((CE_OUT, FE), lambda b, nb: (0, 0)),
        ],
        out_specs=[
            pl.BlockSpec((1, C_OUT, BN), lambda b, nb: (b, 0, nb)),
            pl.BlockSpec((1, CE_OUT, BN), lambda b, nb: (b, 0, nb)),
            pl.BlockSpec((1, FE, BN), lambda b, nb: (b, 0, nb)),
            pl.BlockSpec((2, C_OUT), lambda b, nb: (0, 0)),
            pl.BlockSpec((2, CE_OUT), lambda b, nb: (0, 0)),
        ],
        out_shape=[
            jax.ShapeDtypeStruct((B, C_OUT, N), jnp.float32),
            jax.ShapeDtypeStruct((B, CE_OUT, N), jnp.float32),
            jax.ShapeDtypeStruct((B, FE, N), jnp.float32),
            jax.ShapeDtypeStruct((2, C_OUT), jnp.float32),
            jax.ShapeDtypeStruct((2, CE_OUT), jnp.float32),
        ],
        scratch_shapes=[pltpu.VMEM((2, D, N), jnp.float32)],
        interpret=interpret,
    )(features, idx_t, ef_t, W0, We0)

    out, out_ef = pl.pallas_call(
        _stage2_body,
        out_shape=[
            jax.ShapeDtypeStruct((B, C_OUT, N), jnp.float32),
            jax.ShapeDtypeStruct((B, CE_OUT, N), jnp.float32),
        ],
        interpret=interpret,
    )(features, ymax, zmax, efmax, ystats, zstats, Wsc_ef)
    return out, out_ef


def _ef_xla(edge_list, edge_features, idx):
    src = edge_list[:, 0, :]
    dst = edge_list[:, 1, :]
    key_e = src * N + dst
    eft = jnp.transpose(edge_features, (0, 2, 1))
    dense = jnp.zeros((B, N * N, FE), jnp.float32)
    dense = jax.vmap(lambda d, k, v: d.at[k].add(v))(dense, key_e, eft)
    key_q = jnp.arange(N)[None, :, None] * N + idx
    ef = jax.vmap(lambda d, k: d[k.reshape(-1)])(dense, key_q)
    ef = ef.reshape(B, N, K, FE)
    return jnp.transpose(ef, (0, 3, 2, 1))


@jax.jit
def kernel(points, features, edge_list, edge_features, idx, W0, We0, Wsc_ef):
    del points
    ef_t = _ef_xla(edge_list, edge_features, idx)
    idx_t = jnp.transpose(idx, (0, 2, 1))
    return _dense_stages(features, idx_t, ef_t, W0, We0, Wsc_ef)
